# baseline (speedup 1.0000x reference)
import functools

import jax
import jax.numpy as jnp
from jax import lax
from jax.experimental import pallas as pl
from jax.experimental.pallas import tpu as pltpu
from jax.experimental.pallas import tpu_sc as plsc

VOCAB_SIZE = 100000
D_MODEL = 1024
MAX_LEN = 8192
BATCH = 4
SEQ_LEN = 4096

NUM_CORES = 2
NUM_SUBCORES = 16
NUM_WORKERS = NUM_CORES * NUM_SUBCORES
N_ROWS = BATCH * SEQ_LEN
S_BLOCK = SEQ_LEN // NUM_WORKERS
CHUNK = 8
GROW = BATCH * CHUNK
N_PCHUNKS = S_BLOCK // CHUNK
ROWS_PER_WORKER = BATCH * S_BLOCK
LANES = 16
GROUPS = D_MODEL // LANES
NSETS = 3
LAST_C = N_PCHUNKS - 1


def _body(x_hbm, tok_hbm, pos_hbm, out_hbm, idx_v, toks, poss, sgs, sps, sos):
    wid = lax.axis_index("s") * NUM_CORES + lax.axis_index("c")
    s_base = wid * S_BLOCK

    pltpu.sync_copy(x_hbm.at[pl.ds(wid * ROWS_PER_WORKER, ROWS_PER_WORKER)], idx_v)

    def start_gather(c, s):
        pltpu.async_copy(tok_hbm.at[idx_v.at[pl.ds(c * GROW, GROW)]],
                         toks[s], sgs[s])

    def wait_gather(s):
        pltpu.make_async_copy(tok_hbm.at[pl.ds(0, GROW)], toks[s], sgs[s]).wait()

    def start_pos(c, s):
        pltpu.async_copy(pos_hbm.at[pl.ds(s_base + c * CHUNK, CHUNK)],
                         poss[s], sps[s])

    def wait_pos(s):
        pltpu.make_async_copy(pos_hbm.at[pl.ds(0, CHUNK)], poss[s], sps[s]).wait()

    def start_outs(c, s):
        for b in range(BATCH):
            pltpu.async_copy(
                toks[s].at[pl.ds(b * CHUNK, CHUNK)],
                out_hbm.at[pl.ds(b * SEQ_LEN + s_base + c * CHUNK, CHUNK)],
                sos[s])

    def drain_outs(s):
        for _ in range(BATCH):
            pltpu.make_async_copy(toks[s].at[pl.ds(0, CHUNK)],
                                  out_hbm.at[pl.ds(0, CHUNK)], sos[s]).wait()

    def add_chunk(s):
        tok_v, pos_v = toks[s], poss[s]

        def row_add(r, _):
            for g in range(GROUPS):
                sl = pl.ds(g * LANES, LANES)
                p = pos_v[r, sl]
                for b in range(BATCH):
                    tok_v[b * CHUNK + r, sl] = tok_v[b * CHUNK + r, sl] + p
            return 0

        lax.fori_loop(0, CHUNK, row_add, 0, unroll=False)

    def consume_chunk(c, s, nxt, *, drain_guard, issue_next):
        if drain_guard is None:
            drain_outs(nxt)
        else:
            @pl.when(drain_guard)
            def _():
                drain_outs(nxt)
        if issue_next:
            start_gather(c + 1, nxt)
            start_pos(c + 1, nxt)
        wait_pos(s)
        wait_gather(s)
        add_chunk(s)
        start_outs(c, s)

    start_pos(0, 0)
    start_gather(0, 0)

    @pl.loop(0, N_PCHUNKS - 1, step=NSETS)
    def _chunk_triple(i):
        for cc in range(NSETS):
            c = i + cc
            s = cc
            nxt = (cc + 1) % NSETS
            guard = (i > 0) if cc < 2 else None
            consume_chunk(c, s, nxt, drain_guard=guard, issue_next=True)

    consume_chunk(LAST_C, LAST_C % NSETS, (LAST_C + 1) % NSETS,
                  drain_guard=None, issue_next=False)

    drain_outs((LAST_C - 1) % NSETS)
    drain_outs(LAST_C % NSETS)


@jax.jit
def _run(x_r, token_table, position_table):
    mesh = plsc.VectorSubcoreMesh(core_axis_name="c", subcore_axis_name="s")
    k = pl.kernel(
        _body,
        out_type=jax.ShapeDtypeStruct((N_ROWS, D_MODEL), jnp.float32),
        mesh=mesh,
        scratch_types=[
            pltpu.VMEM((ROWS_PER_WORKER,), jnp.int32),
            [pltpu.VMEM((GROW, D_MODEL), jnp.float32) for _ in range(NSETS)],
            [pltpu.VMEM((CHUNK, D_MODEL), jnp.float32) for _ in range(NSETS)],
            [pltpu.SemaphoreType.DMA for _ in range(NSETS)],
            [pltpu.SemaphoreType.DMA for _ in range(NSETS)],
            [pltpu.SemaphoreType.DMA for _ in range(NSETS)],
        ],
    )
    return k(x_r, token_table, position_table)


def kernel(x, token_table, position_table):
    x_r = (x.astype(jnp.int32)
           .reshape(BATCH, NUM_WORKERS, N_PCHUNKS, CHUNK)
           .transpose(1, 2, 0, 3)
           .reshape(N_ROWS))
    out = _run(x_r, token_table, position_table)
    return out.reshape(BATCH, SEQ_LEN, D_MODEL)


# speedup vs baseline: 1.0209x; 1.0209x over previous
import functools

import jax
import jax.numpy as jnp
from jax import lax
from jax.experimental import pallas as pl
from jax.experimental.pallas import tpu as pltpu
from jax.experimental.pallas import tpu_sc as plsc

VOCAB_SIZE = 100000
D_MODEL = 1024
MAX_LEN = 8192
BATCH = 4
SEQ_LEN = 4096

NUM_CORES = 2
NUM_SUBCORES = 16
NUM_WORKERS = NUM_CORES * NUM_SUBCORES
N_ROWS = BATCH * SEQ_LEN
S_BLOCK = SEQ_LEN // NUM_WORKERS
CHUNK = 8
GROW = BATCH * CHUNK
HALF = GROW // 2
N_PCHUNKS = S_BLOCK // CHUNK
ROWS_PER_WORKER = BATCH * S_BLOCK
LANES = 16
GROUPS = D_MODEL // LANES
LAST_I = N_PCHUNKS - 2


def _body(x_hbm, tok_hbm, pos_hbm, out_hbm, idx_v, toks, poss, sgs, sps, sos):
    wid = lax.axis_index("s") * NUM_CORES + lax.axis_index("c")
    s_base = wid * S_BLOCK

    pltpu.sync_copy(x_hbm.at[pl.ds(wid * ROWS_PER_WORKER, ROWS_PER_WORKER)], idx_v)

    def start_gather(c, s):
        pltpu.async_copy(tok_hbm.at[idx_v.at[pl.ds(c * GROW, HALF)]],
                         toks[2 * s], sgs[s])
        pltpu.async_copy(tok_hbm.at[idx_v.at[pl.ds(c * GROW + HALF, HALF)]],
                         toks[2 * s + 1], sgs[s])

    def wait_gather(s):
        for h in range(2):
            pltpu.make_async_copy(tok_hbm.at[pl.ds(0, HALF)],
                                  toks[2 * s + h], sgs[s]).wait()

    def start_pos(c, pb):
        pltpu.async_copy(pos_hbm.at[pl.ds(s_base + c * CHUNK, CHUNK)],
                         poss[pb], sps[pb])

    def wait_pos(pb):
        pltpu.make_async_copy(pos_hbm.at[pl.ds(0, CHUNK)], poss[pb], sps[pb]).wait()

    def start_outs(c, s):
        for b in range(BATCH):
            pltpu.async_copy(
                toks[2 * s + b // 2].at[pl.ds((b % 2) * CHUNK, CHUNK)],
                out_hbm.at[pl.ds(b * SEQ_LEN + s_base + c * CHUNK, CHUNK)],
                sos[s])

    def drain_outs(s):
        for b in range(BATCH):
            pltpu.make_async_copy(toks[2 * s].at[pl.ds(0, CHUNK)],
                                  out_hbm.at[pl.ds(0, CHUNK)], sos[s]).wait()

    def add_chunk(s, pb):
        pos_v = poss[pb]

        def row_add(r, _):
            for g in range(GROUPS):
                sl = pl.ds(g * LANES, LANES)
                p = pos_v[r, sl]
                for b in range(BATCH):
                    tok_v = toks[2 * s + b // 2]
                    row = (b % 2) * CHUNK + r
                    tok_v[row, sl] = tok_v[row, sl] + p
            return 0

        lax.fori_loop(0, CHUNK, row_add, 0, unroll=False)

    start_pos(0, 0)
    start_gather(0, 0)

    @pl.loop(0, N_PCHUNKS, step=2)
    def _chunk_pair(i):
        for cc in (0, 1):
            c = i + cc
            pb = cc
            s = cc
            o = 1 - cc
            if cc == 0:
                start_pos(c + 1, 1 - pb)
            else:
                @pl.when(i < LAST_I)
                def _():
                    start_pos(c + 1, 1 - pb)

            if cc == 0:
                @pl.when(i > 0)
                def _():
                    drain_outs(o)

                start_gather(c + 1, o)
            else:
                drain_outs(o)

                @pl.when(i < LAST_I)
                def _():
                    start_gather(c + 1, o)

            wait_pos(pb)
            wait_gather(s)
            add_chunk(s, pb)
            start_outs(c, s)

    drain_outs(1)


@jax.jit
def _run(x_r, token_table, position_table):
    mesh = plsc.VectorSubcoreMesh(core_axis_name="c", subcore_axis_name="s")
    k = pl.kernel(
        _body,
        out_type=jax.ShapeDtypeStruct((N_ROWS, D_MODEL), jnp.float32),
        mesh=mesh,
        scratch_types=[
            pltpu.VMEM((ROWS_PER_WORKER,), jnp.int32),
            [pltpu.VMEM((HALF, D_MODEL), jnp.float32) for _ in range(4)],
            [pltpu.VMEM((CHUNK, D_MODEL), jnp.float32) for _ in range(2)],
            [pltpu.SemaphoreType.DMA for _ in range(2)],
            [pltpu.SemaphoreType.DMA for _ in range(2)],
            [pltpu.SemaphoreType.DMA for _ in range(2)],
        ],
    )
    return k(x_r, token_table, position_table)


def kernel(x, token_table, position_table):
    x_r = (x.astype(jnp.int32)
           .reshape(BATCH, NUM_WORKERS, N_PCHUNKS, CHUNK)
           .transpose(1, 2, 0, 3)
           .reshape(N_ROWS))
    out = _run(x_r, token_table, position_table)
    return out.reshape(BATCH, SEQ_LEN, D_MODEL)
